# Initial kernel scaffold; baseline (speedup 1.0000x reference)
import jax
import jax.numpy as jnp
from jax.experimental import pallas as pl


def kernel(x, W_qkv, b_qkv, W_lepe, b_lepe, W_out, b_out):
    raise NotImplementedError("write your pallas kernel here")


# speedup vs baseline: 2.1068x; 2.1068x over previous
import functools

import jax
import jax.numpy as jnp
from jax.experimental import pallas as pl
from jax.experimental.pallas import tpu as pltpu

DIM = 128
NUM_HEADS = 8
N_WIN = 7
TOPK = 4
HEAD_DIM = DIM // NUM_HEADS
SCALE = DIM ** -0.5
NREG = N_WIN ** 3
RSS = 64
SEQ = NREG * RSS
KV = TOPK * RSS

_INTERPRET = False


_QKV_RBLK = 49


def _qkv_kernel(x_ref, w_ref, b_ref, y_ref, pool_ref):
    y = jnp.dot(x_ref[:], w_ref[:], preferred_element_type=jnp.float32) + b_ref[:]
    y_ref[:] = y
    p = y[:, : 2 * DIM].reshape(_QKV_RBLK, RSS, 2 * DIM)
    pool_ref[0] = jnp.sum(p, axis=1) * (1.0 / RSS)


def _qkv_call(x_seq, w_t, b2d):
    m_blk = _QKV_RBLK * RSS
    return pl.pallas_call(
        _qkv_kernel,
        grid=(NREG // _QKV_RBLK,),
        in_specs=[
            pl.BlockSpec((m_blk, DIM), lambda i: (i, 0)),
            pl.BlockSpec((DIM, 3 * DIM), lambda i: (0, 0)),
            pl.BlockSpec((1, 3 * DIM), lambda i: (0, 0)),
        ],
        out_specs=[
            pl.BlockSpec((m_blk, 3 * DIM), lambda i: (i, 0)),
            pl.BlockSpec((1, _QKV_RBLK, 2 * DIM), lambda i: (i, 0, 0)),
        ],
        out_shape=[
            jax.ShapeDtypeStruct((SEQ, 3 * DIM), jnp.float32),
            jax.ShapeDtypeStruct((NREG // _QKV_RBLK, _QKV_RBLK, 2 * DIM), jnp.float32),
        ],
        interpret=_INTERPRET,
    )(x_seq, w_t, b2d)


def _route_kernel(pool_ref, idx_ref):
    qp = pool_ref[:, :DIM]
    kp = pool_ref[:, DIM:]
    a = jax.lax.dot_general(qp, kp, (((1,), (1,)), ((), ())),
                            preferred_element_type=jnp.float32)
    col = jax.lax.broadcasted_iota(jnp.int32, a.shape, 1)
    for j in range(TOPK):
        m = jnp.max(a, axis=1, keepdims=True)
        cand = jnp.where(a >= m, col, NREG + 1)
        sel = jnp.min(cand, axis=1, keepdims=True)
        idx_ref[:, j:j + 1] = sel
        a = jnp.where(col == sel, -jnp.inf, a)


def _route_call(pools):
    return pl.pallas_call(
        _route_kernel,
        out_shape=jax.ShapeDtypeStruct((NREG, TOPK), jnp.int32),
        interpret=_INTERPRET,
    )(pools)


def _attn_kernel(idx_ref, q_ref, k_ref, v_ref, o_ref):
    r = pl.program_id(0)
    q = q_ref[:] * SCALE
    ks = [k_ref[pl.ds(idx_ref[r, j] * RSS, RSS), :] for j in range(TOPK)]
    vs = [v_ref[pl.ds(idx_ref[r, j] * RSS, RSS), :] for j in range(TOPK)]
    kg = jnp.concatenate(ks, axis=0)
    vg = jnp.concatenate(vs, axis=0)
    big = NUM_HEADS * RSS
    qt = jnp.broadcast_to(q[None], (NUM_HEADS, RSS, DIM)).reshape(big, DIM)
    rowh = jax.lax.broadcasted_iota(jnp.int32, (big, DIM), 0) // RSS
    colh = jax.lax.broadcasted_iota(jnp.int32, (big, DIM), 1) // HEAD_DIM
    qbd = jnp.where(rowh == colh, qt, 0.0)
    s = jax.lax.dot_general(qbd, kg, (((1,), (1,)), ((), ())),
                            preferred_element_type=jnp.float32)
    s = s - jnp.max(s, axis=1, keepdims=True)
    e = jnp.exp(s)
    p = e / jnp.sum(e, axis=1, keepdims=True)
    o3 = jnp.dot(p, vg, preferred_element_type=jnp.float32)
    hsel = jax.lax.broadcasted_iota(jnp.int32, (RSS, DIM), 1) // HEAD_DIM
    acc = jnp.zeros((RSS, DIM), jnp.float32)
    for m in range(NUM_HEADS):
        acc = acc + jnp.where(hsel == m, o3[m * RSS:(m + 1) * RSS, :], 0.0)
    o_ref[:] = acc


def _attn_call(q, k, v, idx):
    return pl.pallas_call(
        _attn_kernel,
        grid=(NREG,),
        in_specs=[
            pl.BlockSpec(memory_space=pltpu.SMEM),
            pl.BlockSpec((RSS, DIM), lambda r: (r, 0)),
            pl.BlockSpec((SEQ, DIM), lambda r: (0, 0)),
            pl.BlockSpec((SEQ, DIM), lambda r: (0, 0)),
        ],
        out_specs=pl.BlockSpec((RSS, DIM), lambda r: (r, 0)),
        out_shape=jax.ShapeDtypeStruct((SEQ, DIM), jnp.float32),
        interpret=_INTERPRET,
    )(idx, q, k, v)


def _lepe_kernel(vp_ref, w_ref, b_ref, o_ref):
    acc = jnp.zeros((28, 28, 28, DIM), jnp.float32) + b_ref[:].reshape(1, 1, 1, DIM)
    for t in range(27):
        i, j, k = t // 9, (t // 3) % 3, t % 3
        w_t = w_ref[t:t + 1, :].reshape(1, 1, 1, DIM)
        acc = acc + vp_ref[i:i + 28, j:j + 28, k:k + 28, :] * w_t
    o_ref[:] = acc


def _lepe_call(v_pad, w27, b_lepe):
    return pl.pallas_call(
        _lepe_kernel,
        out_shape=jax.ShapeDtypeStruct((28, 28, 28, DIM), jnp.float32),
        interpret=_INTERPRET,
    )(v_pad, w27, b_lepe)


_PROJ_MBLK = 2744


def _proj_kernel(a_ref, l_ref, w_ref, b_ref, o_ref):
    s = a_ref[:] + l_ref[:]
    o_ref[:] = jnp.dot(s, w_ref[:], preferred_element_type=jnp.float32) + b_ref[:]


def _proj_call(attn_flat, lepe_flat, w_t, b2d):
    return pl.pallas_call(
        _proj_kernel,
        grid=(SEQ // _PROJ_MBLK,),
        in_specs=[
            pl.BlockSpec((_PROJ_MBLK, DIM), lambda i: (i, 0)),
            pl.BlockSpec((_PROJ_MBLK, DIM), lambda i: (i, 0)),
            pl.BlockSpec((DIM, DIM), lambda i: (0, 0)),
            pl.BlockSpec((1, DIM), lambda i: (0, 0)),
        ],
        out_specs=pl.BlockSpec((_PROJ_MBLK, DIM), lambda i: (i, 0)),
        out_shape=jax.ShapeDtypeStruct((SEQ, DIM), jnp.float32),
        interpret=_INTERPRET,
    )(attn_flat, lepe_flat, w_t, b2d)


def kernel(x, W_qkv, b_qkv, W_lepe, b_lepe, W_out, b_out):
    C, H, W_, D = DIM, 28, 28, 28
    rs = H // N_WIN
    xt = x[0].reshape(C, N_WIN, rs, N_WIN, rs, N_WIN, rs)
    xt = jnp.transpose(xt, (1, 3, 5, 2, 4, 6, 0)).reshape(SEQ, C)

    qkv, pools = _qkv_call(xt, W_qkv.T, b_qkv[None, :])
    idx = _route_call(pools.reshape(NREG, 2 * C))
    q = qkv[:, :C]
    k = qkv[:, C:2 * C]
    v = qkv[:, 2 * C:]
    attn_seq = _attn_call(q, k, v, idx)

    def seq2grid_cl(t):
        t = t.reshape(N_WIN, N_WIN, N_WIN, rs, rs, rs, C)
        t = jnp.transpose(t, (0, 3, 1, 4, 2, 5, 6))
        return t.reshape(H, W_, D, C)

    v_pad = jnp.pad(seq2grid_cl(v), ((1, 1), (1, 1), (1, 1), (0, 0)))
    lepe = _lepe_call(v_pad, W_lepe.reshape(C, 27).T, b_lepe[None, :])
    lepe_flat = lepe.reshape(SEQ, C)
    attn_flat = seq2grid_cl(attn_seq).reshape(SEQ, C)

    out_flat = _proj_call(attn_flat, lepe_flat, W_out.T, b_out[None, :])
    out = jnp.transpose(out_flat.reshape(H, W_, D, C), (3, 0, 1, 2))
    return out[None]


# speedup vs baseline: 2.1602x; 1.0253x over previous
import functools

import jax
import jax.numpy as jnp
from jax.experimental import pallas as pl
from jax.experimental.pallas import tpu as pltpu

DIM = 128
NUM_HEADS = 8
N_WIN = 7
TOPK = 4
HEAD_DIM = DIM // NUM_HEADS
SCALE = DIM ** -0.5
NREG = N_WIN ** 3
RSS = 64
SEQ = NREG * RSS
KV = TOPK * RSS

_INTERPRET = False


_QKV_RBLK = 49


def _qkv_kernel(x_ref, w_ref, b_ref, y_ref, pool_ref):
    y = jnp.dot(x_ref[:], w_ref[:], preferred_element_type=jnp.float32) + b_ref[:]
    y_ref[:] = y
    p = y[:, : 2 * DIM].reshape(_QKV_RBLK, RSS, 2 * DIM)
    pool_ref[0] = jnp.sum(p, axis=1) * (1.0 / RSS)


def _qkv_call(x_seq, w_t, b2d):
    m_blk = _QKV_RBLK * RSS
    return pl.pallas_call(
        _qkv_kernel,
        grid=(NREG // _QKV_RBLK,),
        in_specs=[
            pl.BlockSpec((m_blk, DIM), lambda i: (i, 0)),
            pl.BlockSpec((DIM, 3 * DIM), lambda i: (0, 0)),
            pl.BlockSpec((1, 3 * DIM), lambda i: (0, 0)),
        ],
        out_specs=[
            pl.BlockSpec((m_blk, 3 * DIM), lambda i: (i, 0)),
            pl.BlockSpec((1, _QKV_RBLK, 2 * DIM), lambda i: (i, 0, 0)),
        ],
        out_shape=[
            jax.ShapeDtypeStruct((SEQ, 3 * DIM), jnp.float32),
            jax.ShapeDtypeStruct((NREG // _QKV_RBLK, _QKV_RBLK, 2 * DIM), jnp.float32),
        ],
        interpret=_INTERPRET,
    )(x_seq, w_t, b2d)


def _route_kernel(pool_ref, idx_ref):
    qp = pool_ref[:, :DIM]
    kp = pool_ref[:, DIM:]
    a = jax.lax.dot_general(qp, kp, (((1,), (1,)), ((), ())),
                            preferred_element_type=jnp.float32)
    col = jax.lax.broadcasted_iota(jnp.int32, a.shape, 1)
    for j in range(TOPK):
        m = jnp.max(a, axis=1, keepdims=True)
        cand = jnp.where(a >= m, col, NREG + 1)
        sel = jnp.min(cand, axis=1, keepdims=True)
        idx_ref[:, j:j + 1] = sel
        a = jnp.where(col == sel, -jnp.inf, a)


def _route_call(pools):
    return pl.pallas_call(
        _route_kernel,
        out_shape=jax.ShapeDtypeStruct((NREG, TOPK), jnp.int32),
        interpret=_INTERPRET,
    )(pools)


def _attn_kernel(idx_ref, q_ref, k_ref, v_ref, o_ref):
    r = pl.program_id(0)
    q = q_ref[:] * SCALE
    ks = [k_ref[pl.ds(idx_ref[r, j] * RSS, RSS), :] for j in range(TOPK)]
    vs = [v_ref[pl.ds(idx_ref[r, j] * RSS, RSS), :] for j in range(TOPK)]
    kg = jnp.concatenate(ks, axis=0)
    vg = jnp.concatenate(vs, axis=0)
    big = NUM_HEADS * RSS
    qt = jnp.broadcast_to(q[None], (NUM_HEADS, RSS, DIM)).reshape(big, DIM)
    rowh = jax.lax.broadcasted_iota(jnp.int32, (big, DIM), 0) // RSS
    colh = jax.lax.broadcasted_iota(jnp.int32, (big, DIM), 1) // HEAD_DIM
    qbd = jnp.where(rowh == colh, qt, 0.0).astype(jnp.bfloat16)
    s = jax.lax.dot_general(qbd, kg.astype(jnp.bfloat16),
                            (((1,), (1,)), ((), ())),
                            preferred_element_type=jnp.float32)
    s = s - jnp.max(s, axis=1, keepdims=True)
    e = jnp.exp(s)
    denom = jnp.sum(e, axis=1, keepdims=True)
    o3 = jnp.dot(e.astype(jnp.bfloat16), vg.astype(jnp.bfloat16),
                 preferred_element_type=jnp.float32)
    o3 = o3 / denom
    hsel = jax.lax.broadcasted_iota(jnp.int32, (RSS, DIM), 1) // HEAD_DIM
    acc = jnp.zeros((RSS, DIM), jnp.float32)
    for m in range(NUM_HEADS):
        acc = acc + jnp.where(hsel == m, o3[m * RSS:(m + 1) * RSS, :], 0.0)
    o_ref[:] = acc


def _attn_call(q, k, v, idx):
    return pl.pallas_call(
        _attn_kernel,
        grid=(NREG,),
        in_specs=[
            pl.BlockSpec(memory_space=pltpu.SMEM),
            pl.BlockSpec((RSS, DIM), lambda r: (r, 0)),
            pl.BlockSpec((SEQ, DIM), lambda r: (0, 0)),
            pl.BlockSpec((SEQ, DIM), lambda r: (0, 0)),
        ],
        out_specs=pl.BlockSpec((RSS, DIM), lambda r: (r, 0)),
        out_shape=jax.ShapeDtypeStruct((SEQ, DIM), jnp.float32),
        interpret=_INTERPRET,
    )(idx, q, k, v)


def _lepe_kernel(vp_ref, w_ref, b_ref, o_ref):
    acc = jnp.zeros((28, 28, 28, DIM), jnp.float32) + b_ref[:].reshape(1, 1, 1, DIM)
    for t in range(27):
        i, j, k = t // 9, (t // 3) % 3, t % 3
        w_t = w_ref[t:t + 1, :].reshape(1, 1, 1, DIM)
        acc = acc + vp_ref[i:i + 28, j:j + 28, k:k + 28, :] * w_t
    o_ref[:] = acc


def _lepe_call(v_pad, w27, b_lepe):
    return pl.pallas_call(
        _lepe_kernel,
        out_shape=jax.ShapeDtypeStruct((28, 28, 28, DIM), jnp.float32),
        interpret=_INTERPRET,
    )(v_pad, w27, b_lepe)


_PROJ_MBLK = 2744


def _proj_kernel(a_ref, l_ref, w_ref, b_ref, o_ref):
    s = a_ref[:] + l_ref[:]
    o_ref[:] = jnp.dot(s, w_ref[:], preferred_element_type=jnp.float32) + b_ref[:]


def _proj_call(attn_flat, lepe_flat, w_t, b2d):
    return pl.pallas_call(
        _proj_kernel,
        grid=(SEQ // _PROJ_MBLK,),
        in_specs=[
            pl.BlockSpec((_PROJ_MBLK, DIM), lambda i: (i, 0)),
            pl.BlockSpec((_PROJ_MBLK, DIM), lambda i: (i, 0)),
            pl.BlockSpec((DIM, DIM), lambda i: (0, 0)),
            pl.BlockSpec((1, DIM), lambda i: (0, 0)),
        ],
        out_specs=pl.BlockSpec((_PROJ_MBLK, DIM), lambda i: (i, 0)),
        out_shape=jax.ShapeDtypeStruct((SEQ, DIM), jnp.float32),
        interpret=_INTERPRET,
    )(attn_flat, lepe_flat, w_t, b2d)


def kernel(x, W_qkv, b_qkv, W_lepe, b_lepe, W_out, b_out):
    C, H, W_, D = DIM, 28, 28, 28
    rs = H // N_WIN
    xt = x[0].reshape(C, N_WIN, rs, N_WIN, rs, N_WIN, rs)
    xt = jnp.transpose(xt, (1, 3, 5, 2, 4, 6, 0)).reshape(SEQ, C)

    qkv, pools = _qkv_call(xt, W_qkv.T, b_qkv[None, :])
    idx = _route_call(pools.reshape(NREG, 2 * C))
    q = qkv[:, :C]
    k = qkv[:, C:2 * C]
    v = qkv[:, 2 * C:]
    attn_seq = _attn_call(q, k, v, idx)

    def seq2grid_cl(t):
        t = t.reshape(N_WIN, N_WIN, N_WIN, rs, rs, rs, C)
        t = jnp.transpose(t, (0, 3, 1, 4, 2, 5, 6))
        return t.reshape(H, W_, D, C)

    v_pad = jnp.pad(seq2grid_cl(v), ((1, 1), (1, 1), (1, 1), (0, 0)))
    lepe = _lepe_call(v_pad, W_lepe.reshape(C, 27).T, b_lepe[None, :])
    lepe_flat = lepe.reshape(SEQ, C)
    attn_flat = seq2grid_cl(attn_seq).reshape(SEQ, C)

    out_flat = _proj_call(attn_flat, lepe_flat, W_out.T, b_out[None, :])
    out = jnp.transpose(out_flat.reshape(H, W_, D, C), (3, 0, 1, 2))
    return out[None]
